# baseline (speedup 1.0000x reference)
import functools

import jax
import jax.numpy as jnp
from jax import lax
from jax.experimental import pallas as pl
from jax.experimental.pallas import tpu as pltpu
from jax.experimental.pallas import tpu_sc as plsc

_B = 4096
_D = 64
_OFF = 100000
_NC, _NS = 2, 16
_NW = _NC * _NS
_BPW = _B // _NW


@functools.lru_cache(maxsize=None)
def _make_sc_gather():
    mesh = plsc.VectorSubcoreMesh(core_axis_name="c", subcore_axis_name="s")

    @functools.partial(
        pl.kernel,
        mesh=mesh,
        compiler_params=pltpu.CompilerParams(use_tc_tiling_on_sc=False,
                                             needs_layout_passes=False),
        out_type=[
            jax.ShapeDtypeStruct((_B, _D), jnp.float32),
            jax.ShapeDtypeStruct((_B, _D), jnp.float32),
            jax.ShapeDtypeStruct((_B,), jnp.float32),
            jax.ShapeDtypeStruct((_B,), jnp.float32),
        ],
        scratch_types=[
            pltpu.VMEM((_BPW,), jnp.int32),
            pltpu.VMEM((_BPW,), jnp.int32),
            pltpu.VMEM((_BPW,), jnp.int32),
            pltpu.VMEM((_BPW,), jnp.int32),
            pltpu.VMEM((_BPW, _D), jnp.float32),
            pltpu.VMEM((_BPW, _D), jnp.float32),
            pltpu.VMEM((_BPW, 16), jnp.float32),
            pltpu.VMEM((_BPW, 16), jnp.float32),
            pltpu.VMEM((_BPW,), jnp.float32),
            pltpu.VMEM((_BPW,), jnp.float32),
            pltpu.SemaphoreType.DMA,
            pltpu.SemaphoreType.DMA,
            pltpu.SemaphoreType.DMA,
            pltpu.SemaphoreType.DMA,
        ],
    )
    def _sc_gather(x1_hbm, x2_hbm, emb_hbm, lin_hbm,
                   e1_out, e2_out, l1_out, l2_out,
                   idx1_v, idx2_v, lrow1_v, lrow2_v, rows1_v, rows2_v,
                   lbuf1_v, lbuf2_v, l1_v, l2_v,
                   sem1, sem2, sem3, sem4):
        wid = lax.axis_index("s") * _NC + lax.axis_index("c")
        base = wid * _BPW
        pltpu.sync_copy(x1_hbm.at[pl.ds(base, _BPW)], idx1_v)
        pltpu.sync_copy(x2_hbm.at[pl.ds(base, _BPW)], idx2_v)
        for i in range(_BPW // 16):
            sl = pl.ds(i * 16, 16)
            idx2_v[sl] = idx2_v[sl] + _OFF
            lrow1_v[sl] = lax.shift_right_logical(idx1_v[sl], 4)
            lrow2_v[sl] = lax.shift_right_logical(idx2_v[sl], 4)
        cp1 = pltpu.async_copy(emb_hbm.at[idx1_v], rows1_v, sem1)
        cp2 = pltpu.async_copy(emb_hbm.at[idx2_v], rows2_v, sem2)
        cp3 = pltpu.async_copy(lin_hbm.at[lrow1_v], lbuf1_v, sem3)
        cp4 = pltpu.async_copy(lin_hbm.at[lrow2_v], lbuf2_v, sem4)
        cp1.wait()
        cp2.wait()
        cp3.wait()
        cp4.wait()
        lane = lax.iota(jnp.int32, 16)
        for i in range(_BPW // 16):
            sl = pl.ds(i * 16, 16)
            rows = lane + (i * 16)
            l1_v[sl] = plsc.load_gather(lbuf1_v, [rows, idx1_v[sl] & 15])
            l2_v[sl] = plsc.load_gather(lbuf2_v, [rows, idx2_v[sl] & 15])
        pltpu.sync_copy(rows1_v, e1_out.at[pl.ds(base, _BPW)])
        pltpu.sync_copy(rows2_v, e2_out.at[pl.ds(base, _BPW)])
        pltpu.sync_copy(l1_v, l1_out.at[pl.ds(base, _BPW)])
        pltpu.sync_copy(l2_v, l2_out.at[pl.ds(base, _BPW)])

    return _sc_gather


def _dense_body(e1_ref, e2_ref, l1_ref, l2_ref, lin_b_ref,
                w1_ref, b1_ref, g1_ref, be1_ref,
                w2_ref, b2_ref, g2_ref, be2_ref,
                w3_ref, b3_ref, out_ref):
    e1 = e1_ref[...]
    e2 = e2_ref[...]

    s = e1 + e2
    fm = 0.5 * jnp.sum(s * s - e1 * e1 - e2 * e2, axis=1, keepdims=True)

    lin = l1_ref[...] + l2_ref[...] + lin_b_ref[...]

    h = (
        jnp.dot(e1, w1_ref[0:_D, :], preferred_element_type=jnp.float32)
        + jnp.dot(e2, w1_ref[_D:2 * _D, :], preferred_element_type=jnp.float32)
        + b1_ref[...]
    )
    m = jnp.mean(h, axis=0, keepdims=True)
    hc = h - m
    v = jnp.mean(hc * hc, axis=0, keepdims=True)
    h = jnp.maximum(hc * lax.rsqrt(v + 1e-5) * g1_ref[...] + be1_ref[...], 0.0)

    h = jnp.dot(h, w2_ref[...], preferred_element_type=jnp.float32) + b2_ref[...]
    m = jnp.mean(h, axis=0, keepdims=True)
    hc = h - m
    v = jnp.mean(hc * hc, axis=0, keepdims=True)
    h = jnp.maximum(hc * lax.rsqrt(v + 1e-5) * g2_ref[...] + be2_ref[...], 0.0)

    o = jnp.dot(h, w3_ref[...], preferred_element_type=jnp.float32) + b3_ref[...]
    z = lin + fm + o
    out_ref[...] = 1.0 / (1.0 + jnp.exp(-z))


_dense = pl.pallas_call(
    _dense_body,
    out_shape=jax.ShapeDtypeStruct((_B, 1), jnp.float32),
)


def kernel(x1, x2, emb_table, lin_w, lin_b,
           W1, b1, g1, be1, W2, b2, g2, be2, W3, b3):
    e1, e2, l1, l2 = _make_sc_gather()(x1, x2, emb_table,
                                       lin_w.reshape(-1, 16))
    out = _dense(
        e1, e2, l1.reshape(_B, 1), l2.reshape(_B, 1), lin_b.reshape(1, 1),
        W1, b1.reshape(1, -1), g1.reshape(1, -1), be1.reshape(1, -1),
        W2, b2.reshape(1, -1), g2.reshape(1, -1), be2.reshape(1, -1),
        W3, b3.reshape(1, 1),
    )
    return out.reshape(_B)
